# baseline (speedup 1.0000x reference)
import jax
import jax.numpy as jnp
from jax import lax
from jax.experimental import pallas as pl
from jax.experimental.pallas import tpu as pltpu
from jax.experimental.pallas import tpu_sc as plsc

PERIODS = 288
WEEKEND = 7
P_DIM = 24
W_DIM = 24
OUT_DIM = P_DIM + W_DIM
PAD_DIM = 128
N_TAB = PERIODS * WEEKEND
N_NODE = 1000
N_BT = 64 * 12
N_ROWS = N_BT * N_NODE
IDX_PAD = 1024
NW = 32
ROWS_W = N_BT // NW
SZ = (512, 488)
CHUNKS = (
    ((0, 128), (128, 128), (256, 128), (384, 128)),
    ((0, 128), (128, 128), (256, 128), (384, 104)),
)
DEPTH = 4
NJ = 2 * ROWS_W // DEPTH


def _build_table_kernel(p_ref, w_ref, o_ref):
    pe = jnp.broadcast_to(p_ref[:][:, None, :], (PERIODS, WEEKEND, P_DIM))
    we = jnp.broadcast_to(w_ref[:][None, :, :], (PERIODS, WEEKEND, W_DIM))
    o_ref[:] = jnp.concatenate([pe, we], axis=-1)


def _build_fused_table(periods_embedding, weekend_embedding):
    fused3 = pl.pallas_call(
        _build_table_kernel,
        out_shape=jax.ShapeDtypeStruct((PERIODS, WEEKEND, OUT_DIM), jnp.float32),
    )(periods_embedding, weekend_embedding)
    return fused3.reshape(N_TAB, OUT_DIM)


def _idx_kernel(x_ref, o_ref):
    p = x_ref[..., 1]
    w = x_ref[..., 2]
    pi = jnp.minimum((p * float(PERIODS)).astype(jnp.int32), PERIODS - 1)
    wi = jnp.minimum(w.astype(jnp.int32), WEEKEND - 1)
    idx = (pi * WEEKEND + wi).reshape(2 * 12, N_NODE)
    pad = jnp.zeros((2 * 12, IDX_PAD - N_NODE), jnp.int32)
    o_ref[:] = jnp.concatenate([idx, pad], axis=-1)


def _build_idx(x):
    return pl.pallas_call(
        _idx_kernel,
        grid=(32,),
        in_specs=[pl.BlockSpec((2, 12, N_NODE, 3), lambda i: (i, 0, 0, 0))],
        out_specs=pl.BlockSpec((2 * 12, IDX_PAD), lambda i: (i, 0)),
        out_shape=jax.ShapeDtypeStruct((N_BT, IDX_PAD), jnp.int32),
    )(x)


def _sc_body(idx_hbm, tab_hbm, out_hbm, tab_sh, idx_v, rows_v, xs, gs, osem):
    sid = lax.axis_index("s")
    wid = sid * 2 + lax.axis_index("c")
    w_row0 = wid * ROWS_W

    @pl.when(sid == 0)
    def _():
        pltpu.sync_copy(tab_hbm, tab_sh)

    plsc.subcore_barrier()

    def fire_idx(j, d, s):
        row = w_row0 + 2 * j + d // 2
        sz = SZ[d % 2]
        pltpu.async_copy(
            idx_hbm.at[row, pl.ds((d % 2) * 512, sz)], idx_v[s].at[pl.ds(0, sz)], xs[s]
        )

    def wait_idx(j, d, s):
        row = w_row0 + 2 * j + d // 2
        sz = SZ[d % 2]
        pltpu.make_async_copy(
            idx_hbm.at[row, pl.ds((d % 2) * 512, sz)], idx_v[s].at[pl.ds(0, sz)], xs[s]
        ).wait()

    def fire_gathers(d, s):
        for off, cnt in CHUNKS[d % 2]:
            pltpu.async_copy(
                tab_sh.at[idx_v[s].at[pl.ds(off, cnt)]],
                rows_v[s].at[pl.ds(off, cnt)],
                gs[s],
            )

    def wait_gathers(d, s):
        for off, cnt in CHUNKS[d % 2]:
            pltpu.make_async_copy(
                tab_sh.at[idx_v[s].at[pl.ds(off, cnt)]],
                rows_v[s].at[pl.ds(off, cnt)],
                gs[s],
            ).wait()

    def fire_out(j, d, s):
        base = (w_row0 + 2 * j + d // 2) * N_NODE + (d % 2) * 512
        sz = SZ[d % 2]
        pltpu.async_copy(
            rows_v[s].at[pl.ds(0, sz)],
            out_hbm.at[pl.ds(base, sz), pl.ds(0, OUT_DIM)],
            osem[s],
        )

    def wait_out(j, d, s):
        base = (w_row0 + 2 * j + d // 2) * N_NODE + (d % 2) * 512
        sz = SZ[d % 2]
        pltpu.make_async_copy(
            rows_v[s].at[pl.ds(0, sz)],
            out_hbm.at[pl.ds(base, sz), pl.ds(0, OUT_DIM)],
            osem[s],
        ).wait()

    fire_idx(0, 0, 0)
    fire_idx(0, 1, 1)
    wait_idx(0, 0, 0)
    fire_gathers(0, 0)
    fire_idx(0, 2, 2)
    wait_idx(0, 1, 1)
    fire_gathers(1, 1)
    fire_idx(0, 3, 3)
    wait_idx(0, 2, 2)
    fire_gathers(2, 2)
    wait_gathers(0, 0)
    fire_out(0, 0, 0)
    fire_idx(1, 0, 0)
    wait_idx(0, 3, 3)
    fire_gathers(3, 3)
    wait_gathers(1, 1)
    fire_out(0, 1, 1)
    fire_idx(1, 1, 1)

    def steady(j, carry):
        for d in range(DEPTH):
            s = d
            s2 = (d + 2) % DEPTH
            wait_idx(j, d, s)
            wait_out(j - 1, d, s)
            fire_gathers(d, s)
            if d < 2:
                wait_gathers(d + 2, s2)
                fire_out(j - 1, d + 2, s2)
                fire_idx(j, d + 2, s2)
            else:
                wait_gathers(d - 2, s2)
                fire_out(j, d - 2, s2)

                @pl.when(4 * j + d + 2 < 4 * NJ)
                def _():
                    fire_idx(j + 1, d - 2, s2)

        return carry

    lax.fori_loop(1, NJ, steady, 0)

    wait_gathers(2, 2)
    fire_out(NJ - 1, 2, 2)
    wait_gathers(3, 3)
    fire_out(NJ - 1, 3, 3)
    for d in range(DEPTH):
        wait_out(NJ - 1, d, d)


@jax.jit
def _encode(idx, fused_table):
    mesh = plsc.VectorSubcoreMesh(core_axis_name="c", subcore_axis_name="s")
    return pl.kernel(
        _sc_body,
        out_type=jax.ShapeDtypeStruct((N_ROWS, PAD_DIM), jnp.float32),
        mesh=mesh,
        compiler_params=pltpu.CompilerParams(
            needs_layout_passes=False, use_tc_tiling_on_sc=False
        ),
        scratch_types=dict(
            tab_sh=pltpu.VMEM_SHARED((N_TAB, OUT_DIM), jnp.float32),
            idx_v=[pltpu.VMEM((512,), jnp.int32) for _ in range(DEPTH)],
            rows_v=[pltpu.VMEM((512, OUT_DIM), jnp.float32) for _ in range(DEPTH)],
            xs=[pltpu.SemaphoreType.DMA for _ in range(DEPTH)],
            gs=[pltpu.SemaphoreType.DMA for _ in range(DEPTH)],
            osem=[pltpu.SemaphoreType.DMA for _ in range(DEPTH)],
        ),
    )(idx, fused_table)


def kernel(x, periods_embedding, weekend_embedding):
    b, t, n, _ = x.shape
    fused = _build_fused_table(periods_embedding, weekend_embedding)
    idx = _build_idx(x)
    out = _encode(idx, fused)
    return out[:, :OUT_DIM].reshape(b, t, n, OUT_DIM)


# speedup vs baseline: 2.5728x; 2.5728x over previous
import jax
import jax.numpy as jnp
from jax import lax
from jax.experimental import pallas as pl
from jax.experimental.pallas import tpu as pltpu
from jax.experimental.pallas import tpu_sc as plsc

PERIODS = 288
WEEKEND = 7
P_DIM = 24
W_DIM = 24
OUT_DIM = P_DIM + W_DIM
PAD_DIM = 128
N_TAB = PERIODS * WEEKEND
N_ROWS = 64 * 12 * 1000
NW = 32
PER_W = N_ROWS // NW
SUB = 96
NSUB = 5
B_BLK = SUB * NSUB
N_BLK = PER_W // B_BLK
DEPTH = 5


def _build_table_kernel(p_ref, w_ref, o_ref):
    pe = jnp.broadcast_to(p_ref[:][:, None, :], (PERIODS, WEEKEND, P_DIM))
    we = jnp.broadcast_to(w_ref[:][None, :, :], (PERIODS, WEEKEND, W_DIM))
    o_ref[:] = jnp.concatenate([pe, we], axis=-1)


def _build_fused_table(periods_embedding, weekend_embedding):
    fused3 = pl.pallas_call(
        _build_table_kernel,
        out_shape=jax.ShapeDtypeStruct((PERIODS, WEEKEND, OUT_DIM), jnp.float32),
    )(periods_embedding, weekend_embedding)
    return fused3.reshape(N_TAB, OUT_DIM)


def _sc_body(idx_hbm, tab_hbm, out_hbm, tab_sh, idx_v, rows_v, xs, gs, osem):
    sid = lax.axis_index("s")
    wid = sid * 2 + lax.axis_index("c")
    w_base = wid * PER_W

    @pl.when(sid == 0)
    def _():
        pltpu.sync_copy(tab_hbm, tab_sh)

    plsc.subcore_barrier()

    def fire_idx(i, s):
        base = w_base + i * B_BLK
        pltpu.async_copy(idx_hbm.at[pl.ds(base, B_BLK)], idx_v[s], xs[s])

    def wait_idx(i, s):
        base = w_base + i * B_BLK
        pltpu.make_async_copy(idx_hbm.at[pl.ds(base, B_BLK)], idx_v[s], xs[s]).wait()

    def fire_gathers(s):
        for r in range(NSUB):
            pltpu.async_copy(
                tab_sh.at[idx_v[s].at[pl.ds(r * SUB, SUB)]],
                rows_v[s].at[pl.ds(r * SUB, SUB)],
                gs[s],
            )

    def wait_gathers(s):
        for r in range(NSUB):
            pltpu.make_async_copy(
                tab_sh.at[idx_v[s].at[pl.ds(r * SUB, SUB)]],
                rows_v[s].at[pl.ds(r * SUB, SUB)],
                gs[s],
            ).wait()

    def fire_out(i, s):
        base = w_base + i * B_BLK
        pltpu.async_copy(
            rows_v[s], out_hbm.at[pl.ds(base, B_BLK), pl.ds(0, OUT_DIM)], osem[s]
        )

    def wait_out(i, s):
        base = w_base + i * B_BLK
        pltpu.make_async_copy(
            rows_v[s], out_hbm.at[pl.ds(base, B_BLK), pl.ds(0, OUT_DIM)], osem[s]
        ).wait()

    for i in range(3):
        fire_idx(i, i)
    for i in range(DEPTH):
        wait_idx(i, i)
        fire_gathers(i)
        if i >= 2:
            wait_gathers(i - 2)
            fire_out(i - 2, i - 2)
        fire_idx(i + 3, (i + 3) % DEPTH)

    def steady(j, carry):
        i0 = DEPTH * j
        for d in range(DEPTH):
            i = i0 + d
            s = d
            s2 = (d - 2) % DEPTH
            wait_idx(i, s)
            wait_out(i - DEPTH, s)
            fire_gathers(s)
            wait_gathers(s2)
            fire_out(i - 2, s2)

            @pl.when(i + 3 < N_BLK)
            def _():
                fire_idx(i + 3, s2)

        return carry

    lax.fori_loop(1, N_BLK // DEPTH, steady, 0)

    for i in (N_BLK - 2, N_BLK - 1):
        wait_gathers(i % DEPTH)
        fire_out(i, i % DEPTH)
    for s in range(DEPTH):
        wait_out(N_BLK - DEPTH + s, s)


@jax.jit
def _encode(idx, fused_table):
    mesh = plsc.VectorSubcoreMesh(core_axis_name="c", subcore_axis_name="s")
    return pl.kernel(
        _sc_body,
        out_type=jax.ShapeDtypeStruct((N_ROWS, PAD_DIM), jnp.float32),
        mesh=mesh,
        compiler_params=pltpu.CompilerParams(
            needs_layout_passes=False, use_tc_tiling_on_sc=False
        ),
        scratch_types=dict(
            tab_sh=pltpu.VMEM_SHARED((N_TAB, OUT_DIM), jnp.float32),
            idx_v=[pltpu.VMEM((B_BLK,), jnp.int32) for _ in range(DEPTH)],
            rows_v=[pltpu.VMEM((B_BLK, OUT_DIM), jnp.float32) for _ in range(DEPTH)],
            xs=[pltpu.SemaphoreType.DMA for _ in range(DEPTH)],
            gs=[pltpu.SemaphoreType.DMA for _ in range(DEPTH)],
            osem=[pltpu.SemaphoreType.DMA for _ in range(DEPTH)],
        ),
    )(idx, fused_table)


def kernel(x, periods_embedding, weekend_embedding):
    b, t, n, _ = x.shape
    fused = _build_fused_table(periods_embedding, weekend_embedding)
    pi = jnp.minimum((x[..., 1] * float(PERIODS)).astype(jnp.int32), PERIODS - 1)
    wi = jnp.minimum(x[..., 2].astype(jnp.int32), WEEKEND - 1)
    idx = (pi * WEEKEND + wi).reshape(-1)
    out = _encode(idx, fused)
    return out[:, :OUT_DIM].reshape(b, t, n, OUT_DIM)
